# baseline (device time: 476926 ns/iter reference)
import jax
import jax.numpy as jnp
from jax import lax
from jax.experimental import pallas as pl
from jax.experimental.pallas import tpu as pltpu

T = 2048
D = 4096
V_HALF = 8192
HR = T // 2
BLK = 64
NC = 16
CROWS = HR // NC
COLT = 512
NCT = V_HALF // COLT


def _exchange_gemm(mineA, x16, W16):

    def body(mineA_ref, x_ref, w_ref, theirs_ref, mineB_ref,
             xb_buf, w_buf, o_buf,
             xb_sem, w_sems, o_sems, send_y, recv_y, send_x, recv_x):
        my_x = lax.axis_index("x")
        my_y = lax.axis_index("y")
        ynbr = (my_x, 1 - my_y)
        xnbr = (1 - my_x, my_y)

        barrier = pltpu.get_barrier_semaphore()
        for nbr in (ynbr, xnbr):
            pl.semaphore_signal(
                barrier, inc=1, device_id=nbr,
                device_id_type=pl.DeviceIdType.MESH,
            )
        pl.semaphore_wait(barrier, 2)

        part0 = my_x * HR
        q0 = (1 - my_x) * HR

        xb_cp = pltpu.make_async_copy(x_ref.at[pl.ds(q0, HR)], xb_buf, xb_sem)
        xb_cp.start()

        directs = []
        for c in range(NC):
            rdma = pltpu.make_async_remote_copy(
                src_ref=mineA_ref.at[pl.ds(c * CROWS, CROWS)],
                dst_ref=theirs_ref.at[pl.ds(part0 + c * CROWS, CROWS)],
                send_sem=send_y.at[c], recv_sem=recv_y.at[c],
                device_id=ynbr, device_id_type=pl.DeviceIdType.MESH,
            )
            rdma.start()
            directs.append(rdma)

        xb_cp.wait()
        w_cps = [None] * NCT
        w_cps[0] = pltpu.make_async_copy(
            w_ref.at[:, pl.ds(0, COLT)], w_buf.at[0], w_sems.at[0])
        w_cps[0].start()
        o_cps = []
        fwds = []
        for j in range(NCT):
            if j + 1 < NCT:
                w_cps[j + 1] = pltpu.make_async_copy(
                    w_ref.at[:, pl.ds((j + 1) * COLT, COLT)],
                    w_buf.at[(j + 1) % 2], w_sems.at[(j + 1) % 2])
                w_cps[j + 1].start()
            w_cps[j].wait()
            if j >= 2:
                o_cps[j - 2].wait()
            acc = jnp.dot(
                xb_buf[...], w_buf[j % 2],
                preferred_element_type=jnp.float32,
            )
            o_buf[j % 2] = acc.astype(jnp.bfloat16)
            o_cp = pltpu.make_async_copy(
                o_buf.at[j % 2],
                mineB_ref.at[:, pl.ds(j * COLT, COLT)], o_sems.at[j % 2])
            o_cp.start()
            o_cps.append(o_cp)

            directs[j].wait_recv()
            rows = pl.ds(part0 + j * CROWS, CROWS)
            fwd = pltpu.make_async_remote_copy(
                src_ref=theirs_ref.at[rows], dst_ref=theirs_ref.at[rows],
                send_sem=send_x.at[j], recv_sem=recv_x.at[j],
                device_id=xnbr, device_id_type=pl.DeviceIdType.MESH,
            )
            fwd.start()
            fwds.append(fwd)

        o_cps[NCT - 2].wait()
        o_cps[NCT - 1].wait()
        for c in range(NC):
            fwds[c].wait_recv()
        for c in range(NC):
            directs[c].wait_send()
            fwds[c].wait_send()

    return pl.pallas_call(
        body,
        out_shape=(
            jax.ShapeDtypeStruct((T, V_HALF), jnp.bfloat16),
            jax.ShapeDtypeStruct((HR, V_HALF), jnp.bfloat16),
        ),
        in_specs=[pl.BlockSpec(memory_space=pl.ANY)] * 3,
        out_specs=(pl.BlockSpec(memory_space=pl.ANY),
                   pl.BlockSpec(memory_space=pl.ANY)),
        scratch_shapes=[
            pltpu.VMEM((HR, D), jnp.bfloat16),
            pltpu.VMEM((2, D, COLT), jnp.bfloat16),
            pltpu.VMEM((2, HR, COLT), jnp.bfloat16),
            pltpu.SemaphoreType.DMA,
            pltpu.SemaphoreType.DMA((2,)),
            pltpu.SemaphoreType.DMA((2,)),
            pltpu.SemaphoreType.DMA((NC,)),
            pltpu.SemaphoreType.DMA((NC,)),
            pltpu.SemaphoreType.DMA((NC,)),
            pltpu.SemaphoreType.DMA((NC,)),
        ],
        compiler_params=pltpu.CompilerParams(collective_id=0),
    )(mineA, x16, W16)


def _softmax_assemble(mine, theirs):

    def body(mine_ref, theirs_ref, out_ref):
        my_y = lax.axis_index("y")
        mn = mine_ref[...].astype(jnp.float32)
        th = theirs_ref[...].astype(jnp.float32)
        m = jnp.maximum(
            mn.max(axis=-1, keepdims=True), th.max(axis=-1, keepdims=True)
        )
        em = jnp.exp(mn - m)
        et = jnp.exp(th - m)
        s = em.sum(axis=-1, keepdims=True) + et.sum(axis=-1, keepdims=True)
        pm = em / s
        pt = et / s

        @pl.when(my_y == 0)
        def _():
            out_ref[:, :V_HALF] = pm
            out_ref[:, V_HALF:] = pt

        @pl.when(my_y == 1)
        def _():
            out_ref[:, :V_HALF] = pt
            out_ref[:, V_HALF:] = pm

    return pl.pallas_call(
        body,
        out_shape=jax.ShapeDtypeStruct((T, 2 * V_HALF), jnp.float32),
        grid=(T // BLK,),
        in_specs=[
            pl.BlockSpec((BLK, V_HALF), lambda i: (i, 0)),
            pl.BlockSpec((BLK, V_HALF), lambda i: (i, 0)),
        ],
        out_specs=pl.BlockSpec((BLK, 2 * V_HALF), lambda i: (i, 0)),
    )(mine, theirs)


def kernel(x, W):
    my_x = lax.axis_index("x")
    x16 = x.astype(jnp.bfloat16)
    W16 = W.astype(jnp.bfloat16)
    xA = lax.dynamic_slice(x16, (my_x * HR, 0), (HR, D))
    mineA = jnp.dot(xA, W16, preferred_element_type=jnp.bfloat16)
    theirs, mineB = _exchange_gemm(mineA, x16, W16)
    mine = lax.cond(
        my_x == 0,
        lambda a, b: jnp.concatenate([a, b], axis=0),
        lambda a, b: jnp.concatenate([b, a], axis=0),
        mineA, mineB,
    )
    return _softmax_assemble(mine, theirs)


# device time: 476491 ns/iter; 1.0009x vs baseline; 1.0009x over previous
import jax
import jax.numpy as jnp
from jax import lax
from jax.experimental import pallas as pl
from jax.experimental.pallas import tpu as pltpu

T = 2048
D = 4096
V_HALF = 8192
HR = T // 2
BLK = 64
NC = 16
CROWS = HR // NC
COLT = 512
NCT = V_HALF // COLT


def _exchange_gemm(mineA, x16, W16):

    def body(mineA_ref, x_ref, w_ref, theirs_ref, mineB_ref,
             xb_buf, w_buf, o_buf,
             xb_sem, w_sems, o_sems, send_y, recv_y, send_x, recv_x):
        my_x = lax.axis_index("x")
        my_y = lax.axis_index("y")
        ynbr = (my_x, 1 - my_y)
        xnbr = (1 - my_x, my_y)

        barrier = pltpu.get_barrier_semaphore()
        for nbr in (ynbr, xnbr):
            pl.semaphore_signal(
                barrier, inc=1, device_id=nbr,
                device_id_type=pl.DeviceIdType.MESH,
            )
        pl.semaphore_wait(barrier, 2)

        part0 = my_x * HR
        q0 = (1 - my_x) * HR

        xb_cp = pltpu.make_async_copy(x_ref.at[pl.ds(q0, HR)], xb_buf, xb_sem)
        xb_cp.start()

        directs = []
        for c in range(NC):
            rdma = pltpu.make_async_remote_copy(
                src_ref=mineA_ref.at[pl.ds(c * CROWS, CROWS)],
                dst_ref=theirs_ref.at[pl.ds(part0 + c * CROWS, CROWS)],
                send_sem=send_y.at[c], recv_sem=recv_y.at[c],
                device_id=ynbr, device_id_type=pl.DeviceIdType.MESH,
            )
            rdma.start()
            directs.append(rdma)

        xb_cp.wait()
        w_cps = [None] * NCT
        w_cps[0] = pltpu.make_async_copy(
            w_ref.at[:, pl.ds(0, COLT)], w_buf.at[0], w_sems.at[0])
        w_cps[0].start()
        o_cps = []
        fwds = []
        for j in range(NCT):
            if j + 1 < NCT:
                w_cps[j + 1] = pltpu.make_async_copy(
                    w_ref.at[:, pl.ds((j + 1) * COLT, COLT)],
                    w_buf.at[(j + 1) % 2], w_sems.at[(j + 1) % 2])
                w_cps[j + 1].start()

            directs[j].wait_recv()
            rows = pl.ds(part0 + j * CROWS, CROWS)
            fwd = pltpu.make_async_remote_copy(
                src_ref=theirs_ref.at[rows], dst_ref=theirs_ref.at[rows],
                send_sem=send_x.at[j], recv_sem=recv_x.at[j],
                device_id=xnbr, device_id_type=pl.DeviceIdType.MESH,
            )
            fwd.start()
            fwds.append(fwd)

            w_cps[j].wait()
            if j >= 2:
                o_cps[j - 2].wait()
            acc = jnp.dot(
                xb_buf[...], w_buf[j % 2],
                preferred_element_type=jnp.float32,
            )
            o_buf[j % 2] = acc.astype(jnp.bfloat16)
            o_cp = pltpu.make_async_copy(
                o_buf.at[j % 2],
                mineB_ref.at[:, pl.ds(j * COLT, COLT)], o_sems.at[j % 2])
            o_cp.start()
            o_cps.append(o_cp)

        o_cps[NCT - 2].wait()
        o_cps[NCT - 1].wait()
        for c in range(NC):
            fwds[c].wait_recv()
        for c in range(NC):
            directs[c].wait_send()
            fwds[c].wait_send()

    return pl.pallas_call(
        body,
        out_shape=(
            jax.ShapeDtypeStruct((T, V_HALF), jnp.bfloat16),
            jax.ShapeDtypeStruct((HR, V_HALF), jnp.bfloat16),
        ),
        in_specs=[pl.BlockSpec(memory_space=pl.ANY)] * 3,
        out_specs=(pl.BlockSpec(memory_space=pl.ANY),
                   pl.BlockSpec(memory_space=pl.ANY)),
        scratch_shapes=[
            pltpu.VMEM((HR, D), jnp.bfloat16),
            pltpu.VMEM((2, D, COLT), jnp.bfloat16),
            pltpu.VMEM((2, HR, COLT), jnp.bfloat16),
            pltpu.SemaphoreType.DMA,
            pltpu.SemaphoreType.DMA((2,)),
            pltpu.SemaphoreType.DMA((2,)),
            pltpu.SemaphoreType.DMA((NC,)),
            pltpu.SemaphoreType.DMA((NC,)),
            pltpu.SemaphoreType.DMA((NC,)),
            pltpu.SemaphoreType.DMA((NC,)),
        ],
        compiler_params=pltpu.CompilerParams(collective_id=0),
    )(mineA, x16, W16)


def _softmax_assemble(mine, theirs):

    def body(mine_ref, theirs_ref, out_ref):
        my_y = lax.axis_index("y")
        mn = mine_ref[...].astype(jnp.float32)
        th = theirs_ref[...].astype(jnp.float32)
        m = jnp.maximum(
            mn.max(axis=-1, keepdims=True), th.max(axis=-1, keepdims=True)
        )
        em = jnp.exp(mn - m)
        et = jnp.exp(th - m)
        s = em.sum(axis=-1, keepdims=True) + et.sum(axis=-1, keepdims=True)
        pm = em / s
        pt = et / s

        @pl.when(my_y == 0)
        def _():
            out_ref[:, :V_HALF] = pm
            out_ref[:, V_HALF:] = pt

        @pl.when(my_y == 1)
        def _():
            out_ref[:, :V_HALF] = pt
            out_ref[:, V_HALF:] = pm

    return pl.pallas_call(
        body,
        out_shape=jax.ShapeDtypeStruct((T, 2 * V_HALF), jnp.float32),
        grid=(T // BLK,),
        in_specs=[
            pl.BlockSpec((BLK, V_HALF), lambda i: (i, 0)),
            pl.BlockSpec((BLK, V_HALF), lambda i: (i, 0)),
        ],
        out_specs=pl.BlockSpec((BLK, 2 * V_HALF), lambda i: (i, 0)),
    )(mine, theirs)


def kernel(x, W):
    my_x = lax.axis_index("x")
    x16 = x.astype(jnp.bfloat16)
    W16 = W.astype(jnp.bfloat16)
    xA = lax.dynamic_slice(x16, (my_x * HR, 0), (HR, D))
    mineA = jnp.dot(xA, W16, preferred_element_type=jnp.bfloat16)
    theirs, mineB = _exchange_gemm(mineA, x16, W16)
    mine = lax.cond(
        my_x == 0,
        lambda a, b: jnp.concatenate([a, b], axis=0),
        lambda a, b: jnp.concatenate([b, a], axis=0),
        mineA, mineB,
    )
    return _softmax_assemble(mine, theirs)


# device time: 466909 ns/iter; 1.0215x vs baseline; 1.0205x over previous
import jax
import jax.numpy as jnp
from jax import lax
from jax.experimental import pallas as pl
from jax.experimental.pallas import tpu as pltpu

T = 2048
D = 4096
V_HALF = 8192
BLK = 64
NC = 16
CROWS = T // 2 // NC


def _exchange(mine):

    def body(src_ref, out_ref, send_y, recv_y, send_x, recv_x):
        my_x = lax.axis_index("x")
        my_y = lax.axis_index("y")
        ynbr = (my_x, 1 - my_y)
        xnbr = (1 - my_x, my_y)

        barrier = pltpu.get_barrier_semaphore()
        for nbr in (ynbr, xnbr):
            pl.semaphore_signal(
                barrier, inc=1, device_id=nbr,
                device_id_type=pl.DeviceIdType.MESH,
            )
        pl.semaphore_wait(barrier, 2)

        part0 = my_x * (T // 2)

        directs = []
        for c in range(NC):
            rows = pl.ds(part0 + c * CROWS, CROWS)
            rdma = pltpu.make_async_remote_copy(
                src_ref=src_ref.at[rows], dst_ref=out_ref.at[rows],
                send_sem=send_y.at[c], recv_sem=recv_y.at[c],
                device_id=ynbr, device_id_type=pl.DeviceIdType.MESH,
            )
            rdma.start()
            directs.append(rdma)

        fwds = []
        for c in range(NC):
            directs[c].wait_recv()
            rows = pl.ds(part0 + c * CROWS, CROWS)
            fwd = pltpu.make_async_remote_copy(
                src_ref=out_ref.at[rows], dst_ref=out_ref.at[rows],
                send_sem=send_x.at[c], recv_sem=recv_x.at[c],
                device_id=xnbr, device_id_type=pl.DeviceIdType.MESH,
            )
            fwd.start()
            fwds.append(fwd)

        for c in range(NC):
            fwds[c].wait_recv()
        for c in range(NC):
            directs[c].wait_send()
            fwds[c].wait_send()

    return pl.pallas_call(
        body,
        out_shape=jax.ShapeDtypeStruct(mine.shape, mine.dtype),
        in_specs=[pl.BlockSpec(memory_space=pl.ANY)],
        out_specs=pl.BlockSpec(memory_space=pl.ANY),
        scratch_shapes=[pltpu.SemaphoreType.DMA((NC,))] * 4,
        compiler_params=pltpu.CompilerParams(collective_id=0),
    )(mine)


def _softmax_assemble(mine, theirs):

    def body(mine_ref, theirs_ref, out_ref):
        my_y = lax.axis_index("y")
        mn = mine_ref[...].astype(jnp.float32)
        th = theirs_ref[...].astype(jnp.float32)
        m = jnp.maximum(
            mn.max(axis=-1, keepdims=True), th.max(axis=-1, keepdims=True)
        )
        em = jnp.exp(mn - m)
        et = jnp.exp(th - m)
        s = em.sum(axis=-1, keepdims=True) + et.sum(axis=-1, keepdims=True)
        pm = em / s
        pt = et / s

        @pl.when(my_y == 0)
        def _():
            out_ref[:, :V_HALF] = pm
            out_ref[:, V_HALF:] = pt

        @pl.when(my_y == 1)
        def _():
            out_ref[:, :V_HALF] = pt
            out_ref[:, V_HALF:] = pm

    return pl.pallas_call(
        body,
        out_shape=jax.ShapeDtypeStruct((T, 2 * V_HALF), jnp.float32),
        grid=(T // BLK,),
        in_specs=[
            pl.BlockSpec((BLK, V_HALF), lambda i: (i, 0)),
            pl.BlockSpec((BLK, V_HALF), lambda i: (i, 0)),
        ],
        out_specs=pl.BlockSpec((BLK, 2 * V_HALF), lambda i: (i, 0)),
    )(mine, theirs)


def kernel(x, W):
    mine = jnp.dot(
        x.astype(jnp.bfloat16), W.astype(jnp.bfloat16),
        preferred_element_type=jnp.bfloat16,
    )
    theirs = _exchange(mine)
    return _softmax_assemble(mine, theirs)
